# baseline (speedup 1.0000x reference)
import jax
import jax.numpy as jnp
from jax.experimental import pallas as pl
from jax.experimental.pallas import tpu as pltpu

B = 1
S = 2048
HIDDEN = 2048
NH = 16
NKV = 8
G = NH // NKV
HD = 128
EPS = 1e-6
SCALE = HD ** -0.5

BQ = 256
BK = 256
BQ2 = BQ * G


def _rope(x, cos, sin):
    x1 = x[:, : HD // 2]
    x2 = x[:, HD // 2:]
    rot = jnp.concatenate([-x2, x1], axis=1)
    return x * cos + rot * sin


def _fused_kernel(hs_ref, wq_ref, wk_ref, wv_ref, wo_ref, cos_ref, sin_ref,
                  qw_ref, kw_ref, out_ref,
                  ks_ref, vs_ref, qs_ref, attn_ref, acc_ref, l_ref):
    i = pl.program_id(0)

    x = hs_ref[...].astype(jnp.bfloat16)
    cos = cos_ref[...]
    sin = sin_ref[...]
    qw = qw_ref[...]
    kw = kw_ref[...]

    q = jnp.dot(x, wq_ref[...], preferred_element_type=jnp.float32)
    for h in range(NH):
        qh = q[:, h * HD:(h + 1) * HD]
        var = jnp.mean(qh * qh, axis=-1, keepdims=True)
        qh = qh * (jax.lax.rsqrt(var + EPS) * SCALE) * qw
        qs_ref[h // G, (h % G) * BQ:(h % G + 1) * BQ, :] = (
            _rope(qh, cos, sin).astype(jnp.bfloat16))

    k = jnp.dot(x, wk_ref[...], preferred_element_type=jnp.float32)
    for h in range(NKV):
        kh = k[:, h * HD:(h + 1) * HD]
        var = jnp.mean(kh * kh, axis=-1, keepdims=True)
        kh = kh * jax.lax.rsqrt(var + EPS) * kw
        ks_ref[h, pl.ds(i * BQ, BQ), :] = _rope(kh, cos, sin).astype(jnp.bfloat16)

    v = jnp.dot(x, wv_ref[...], preferred_element_type=jnp.float32)
    for h in range(NKV):
        vs_ref[h, pl.ds(i * BQ, BQ), :] = v[:, h * HD:(h + 1) * HD].astype(jnp.bfloat16)

    row = jax.lax.broadcasted_iota(jnp.int32, (BQ2, BK), 0)
    col = jax.lax.broadcasted_iota(jnp.int32, (BQ2, BK), 1)
    diag_mask = col <= jax.lax.rem(row, BQ)
    row2 = jax.lax.broadcasted_iota(jnp.int32, (BQ2, 2 * BK), 0)
    col2 = jax.lax.broadcasted_iota(jnp.int32, (BQ2, 2 * BK), 1)
    diag_mask2 = col2 <= jax.lax.rem(row2, BQ) + BK

    def _step(p_, c0, width, masked, first=False):
        q2 = qs_ref[p_]
        kj = ks_ref[p_, pl.ds(c0, width), :]
        vj = vs_ref[p_, pl.ds(c0, width), :]
        s = jax.lax.dot_general(
            q2, kj, (((1,), (1,)), ((), ())),
            preferred_element_type=jnp.float32)
        p = jnp.exp(s)
        if masked:
            p = jnp.where(diag_mask if width == BK else diag_mask2, p, 0.0)
        lp = jnp.sum(p, axis=1, keepdims=True)
        av = jnp.dot(p.astype(jnp.bfloat16), vj,
                     preferred_element_type=jnp.float32)
        if first:
            l_ref[p_] = lp
            acc_ref[p_] = av
        else:
            l_ref[p_] += lp
            acc_ref[p_] += av

    @pl.when(i % 2 == 0)
    def _():
        for p_ in range(NKV):
            _step(p_, i * BK, BK, masked=True, first=True)

    @pl.when(i % 2 == 1)
    def _():
        for p_ in range(NKV):
            _step(p_, (i - 1) * BK, 2 * BK, masked=True, first=True)

    def body(j, _):
        for p_ in range(NKV):
            _step(p_, j * 2 * BK, 2 * BK, masked=False)
        return 0

    jax.lax.fori_loop(0, i // 2, body, 0)

    for p_ in range(NKV):
        out2 = (acc_ref[p_] / l_ref[p_]).astype(jnp.bfloat16)
        for g in range(G):
            h = G * p_ + g
            attn_ref[:, h * HD:(h + 1) * HD] = out2[g * BQ:(g + 1) * BQ]

    out_ref[...] = jnp.dot(attn_ref[...], wo_ref[...],
                           preferred_element_type=jnp.float32)


@jax.jit
def kernel(hidden_states, cos, sin, Wq, Wk, Wv, Wo, q_norm_w, k_norm_w):
    hs = hidden_states.reshape(S, HIDDEN)
    cos2 = cos.reshape(S, HD)
    sin2 = sin.reshape(S, HD)
    qw = q_norm_w.reshape(1, HD)
    kw = k_norm_w.reshape(1, HD)
    wq16 = Wq.astype(jnp.bfloat16)
    wk16 = Wk.astype(jnp.bfloat16)
    wv16 = Wv.astype(jnp.bfloat16)
    wo16 = Wo.astype(jnp.bfloat16)

    out = pl.pallas_call(
        _fused_kernel,
        grid=(S // BQ,),
        in_specs=[
            pl.BlockSpec((BQ, HIDDEN), lambda i: (i, 0)),
            pl.BlockSpec((HIDDEN, NH * HD), lambda i: (0, 0)),
            pl.BlockSpec((HIDDEN, NKV * HD), lambda i: (0, 0)),
            pl.BlockSpec((HIDDEN, NKV * HD), lambda i: (0, 0)),
            pl.BlockSpec((NH * HD, HIDDEN), lambda i: (0, 0)),
            pl.BlockSpec((BQ, HD), lambda i: (i, 0)),
            pl.BlockSpec((BQ, HD), lambda i: (i, 0)),
            pl.BlockSpec((1, HD), lambda i: (0, 0)),
            pl.BlockSpec((1, HD), lambda i: (0, 0)),
        ],
        out_specs=pl.BlockSpec((BQ, HIDDEN), lambda i: (i, 0)),
        out_shape=jax.ShapeDtypeStruct((S, HIDDEN), jnp.float32),
        scratch_shapes=[
            pltpu.VMEM((NKV, S, HD), jnp.bfloat16),
            pltpu.VMEM((NKV, S, HD), jnp.bfloat16),
            pltpu.VMEM((NKV, BQ2, HD), jnp.bfloat16),
            pltpu.VMEM((BQ, NH * HD), jnp.bfloat16),
            pltpu.VMEM((NKV, BQ2, HD), jnp.float32),
            pltpu.VMEM((NKV, BQ2, 1), jnp.float32),
        ],
    )(hs, wq16, wk16, wv16, wo16, cos2, sin2, qw, kw)

    return out.reshape(B, S, HIDDEN)


# speedup vs baseline: 1.0048x; 1.0048x over previous
import jax
import jax.numpy as jnp
from jax.experimental import pallas as pl
from jax.experimental.pallas import tpu as pltpu

B = 1
S = 2048
HIDDEN = 2048
NH = 16
NKV = 8
G = NH // NKV
HD = 128
EPS = 1e-6
SCALE = HD ** -0.5

BQ = 256
BK = 256
BQ2 = BQ * G


def _rope(x, cos, sin):
    x1 = x[:, : HD // 2]
    x2 = x[:, HD // 2:]
    rot = jnp.concatenate([-x2, x1], axis=1)
    return x * cos + rot * sin


def _fused_kernel(hs_ref, wq_ref, wk_ref, wv_ref, wo_ref, cos_ref, sin_ref,
                  qw_ref, kw_ref, out_ref,
                  ks_ref, vs_ref, qs_ref, attn_ref, acc_ref, l_ref):
    i = pl.program_id(0)

    x = hs_ref[...].astype(jnp.bfloat16)
    cos = cos_ref[...]
    sin = sin_ref[...]
    qw = qw_ref[...]
    kw = kw_ref[...]

    q = jnp.dot(x, wq_ref[...], preferred_element_type=jnp.float32)
    for h in range(NH):
        qh = q[:, h * HD:(h + 1) * HD]
        var = jnp.mean(qh * qh, axis=-1, keepdims=True)
        qh = qh * (jax.lax.rsqrt(var + EPS) * SCALE) * qw
        qs_ref[h // G, (h % G) * BQ:(h % G + 1) * BQ, :] = (
            _rope(qh, cos, sin).astype(jnp.bfloat16))

    k = jnp.dot(x, wk_ref[...], preferred_element_type=jnp.float32)
    for h in range(NKV):
        kh = k[:, h * HD:(h + 1) * HD]
        var = jnp.mean(kh * kh, axis=-1, keepdims=True)
        kh = kh * jax.lax.rsqrt(var + EPS) * kw
        ks_ref[h, pl.ds(i * BQ, BQ), :] = _rope(kh, cos, sin).astype(jnp.bfloat16)

    v = jnp.dot(x, wv_ref[...], preferred_element_type=jnp.float32)
    for h in range(NKV):
        vs_ref[h, pl.ds(i * BQ, BQ), :] = v[:, h * HD:(h + 1) * HD].astype(jnp.bfloat16)

    row = jax.lax.broadcasted_iota(jnp.int32, (BQ2, BK), 0)
    col = jax.lax.broadcasted_iota(jnp.int32, (BQ2, BK), 1)
    diag_mask = col <= jax.lax.rem(row, BQ)

    def _step(p_, c0, width, masked, first=False):
        q2 = qs_ref[p_]
        kj = ks_ref[p_, pl.ds(c0, width), :]
        vj = vs_ref[p_, pl.ds(c0, width), :]
        s = jax.lax.dot_general(
            q2, kj, (((1,), (1,)), ((), ())),
            preferred_element_type=jnp.float32)
        p = jnp.exp(s)
        if masked:
            p = jnp.where(diag_mask, p, 0.0)
        lp = jnp.sum(p, axis=1, keepdims=True)
        av = jnp.dot(p.astype(jnp.bfloat16), vj,
                     preferred_element_type=jnp.float32)
        if first:
            l_ref[p_] = lp
            acc_ref[p_] = av
        else:
            l_ref[p_] += lp
            acc_ref[p_] += av

    for p_ in range(NKV):
        _step(p_, i * BK, BK, masked=True, first=True)

    @pl.when(i % 2 == 1)
    def _():
        for p_ in range(NKV):
            _step(p_, (i - 1) * BK, BK, masked=False)

    def body(j, _):
        for p_ in range(NKV):
            _step(p_, j * 2 * BK, 2 * BK, masked=False)
        return 0

    jax.lax.fori_loop(0, i // 2, body, 0)

    for p_ in range(NKV):
        out2 = (acc_ref[p_] / l_ref[p_]).astype(jnp.bfloat16)
        for g in range(G):
            h = G * p_ + g
            attn_ref[:, h * HD:(h + 1) * HD] = out2[g * BQ:(g + 1) * BQ]

    out_ref[...] = jnp.dot(attn_ref[...], wo_ref[...],
                           preferred_element_type=jnp.float32)


@jax.jit
def kernel(hidden_states, cos, sin, Wq, Wk, Wv, Wo, q_norm_w, k_norm_w):
    hs = hidden_states.reshape(S, HIDDEN)
    cos2 = cos.reshape(S, HD)
    sin2 = sin.reshape(S, HD)
    qw = q_norm_w.reshape(1, HD)
    kw = k_norm_w.reshape(1, HD)
    wq16 = Wq.astype(jnp.bfloat16)
    wk16 = Wk.astype(jnp.bfloat16)
    wv16 = Wv.astype(jnp.bfloat16)
    wo16 = Wo.astype(jnp.bfloat16)

    out = pl.pallas_call(
        _fused_kernel,
        grid=(S // BQ,),
        in_specs=[
            pl.BlockSpec((BQ, HIDDEN), lambda i: (i, 0)),
            pl.BlockSpec((HIDDEN, NH * HD), lambda i: (0, 0)),
            pl.BlockSpec((HIDDEN, NKV * HD), lambda i: (0, 0)),
            pl.BlockSpec((HIDDEN, NKV * HD), lambda i: (0, 0)),
            pl.BlockSpec((NH * HD, HIDDEN), lambda i: (0, 0)),
            pl.BlockSpec((BQ, HD), lambda i: (i, 0)),
            pl.BlockSpec((BQ, HD), lambda i: (i, 0)),
            pl.BlockSpec((1, HD), lambda i: (0, 0)),
            pl.BlockSpec((1, HD), lambda i: (0, 0)),
        ],
        out_specs=pl.BlockSpec((BQ, HIDDEN), lambda i: (i, 0)),
        out_shape=jax.ShapeDtypeStruct((S, HIDDEN), jnp.float32),
        scratch_shapes=[
            pltpu.VMEM((NKV, S, HD), jnp.bfloat16),
            pltpu.VMEM((NKV, S, HD), jnp.bfloat16),
            pltpu.VMEM((NKV, BQ2, HD), jnp.bfloat16),
            pltpu.VMEM((BQ, NH * HD), jnp.bfloat16),
            pltpu.VMEM((NKV, BQ2, HD), jnp.float32),
            pltpu.VMEM((NKV, BQ2, 1), jnp.float32),
        ],
    )(hs, wq16, wk16, wv16, wo16, cos2, sin2, qw, kw)

    return out.reshape(B, S, HIDDEN)
